# baseline (speedup 1.0000x reference)
import jax
import jax.numpy as jnp
from jax.experimental import pallas as pl

_H = 0.3
_DIM = 2
_RHO0 = 1000.0
_C0 = 10.0
_NU = 0.0001
_GAMMA = 7.0
_B = _RHO0 * _C0 ** 2 / _GAMMA
_PI = 3.14159265
_SIGMA = 10.0 / (7.0 * _PI)
_ALPHA = _SIGMA / _H ** _DIM
_INV_H = 1.0 / _H
_GCOEF = _ALPHA / _H ** 2
_CG_A = 2.25 * _GCOEF * _INV_H
_CG_B = -3.0 * _GCOEF
_GRAV_Y = -9.81

_BR = 256
_NB = 16
_NCO = _NB // 2 + 1


def _pressure_from_rho(rho):
    x = rho * (1.0 / _RHO0)
    x2 = x * x
    x3 = x2 * x
    return _B * (x3 * x3 * x - 1.0)


def _pair_geometry(pos_ref, posT_ref):
    x_i = pos_ref[:, 0:1]
    y_i = pos_ref[:, 1:2]
    x_j = posT_ref[0:1, :]
    y_j = posT_ref[1:2, :]
    dx = x_j - x_i
    dy = y_j - y_i
    d2 = dx * dx + dy * dy
    dist = jnp.sqrt(jnp.maximum(d2, 1e-24))
    return dx, dy, d2, dist


def _active(r, c):
    return jnp.logical_not((c == _NCO - 1) & (r >= _NB // 2))


def _density_body(pos_ref, posT_ref, rho_r_ref, rho_c_ref):
    r = pl.program_id(0)
    c = pl.program_id(1)
    _, _, _, dist = _pair_geometry(pos_ref, posT_ref)
    q = jnp.minimum(dist * _INV_H, 2.0)
    q2 = q * q
    w_in = _ALPHA + q2 * ((0.75 * _ALPHA) * q - (1.5 * _ALPHA))
    s = 2.0 - q
    w_out = (0.25 * _ALPHA) * (s * s) * s
    w = jnp.where(q < 1.0, w_in, w_out)
    row_part = jnp.sum(w, axis=1, keepdims=True)

    @pl.when((r == 0) & (c == 0))
    def _():
        rho_c_ref[...] = jnp.zeros_like(rho_c_ref)

    @pl.when(c == 0)
    def _():
        rho_r_ref[...] = row_part

    @pl.when((c > 0) & _active(r, c))
    def _():
        rho_r_ref[...] = rho_r_ref[...] + row_part
        bj = jax.lax.rem(r + c, _NB)
        col_part = jnp.sum(w, axis=0, keepdims=True)
        sl = pl.ds(bj * _BR, _BR)
        rho_c_ref[0:1, sl] = rho_c_ref[0:1, sl] + col_part


def _force_body(pos_ref, vel_ref, posT_ref, velT_ref,
                rho_rs_ref, rho_csT_ref, rho_rsT_ref, rho_cs_ref,
                fr_ref, fc_ref):
    r = pl.program_id(0)
    c = pl.program_id(1)
    dx, dy, d2, dist = _pair_geometry(pos_ref, posT_ref)
    mask = (dist < _H) & (dist > 1e-10)
    cgrad = _CG_A * dist + _CG_B
    cr = cgrad * jnp.minimum(d2 * 1e10, 1.0)

    rho_i = jnp.maximum(rho_rs_ref[...] + rho_csT_ref[...], 0.0001)
    rho_j = jnp.maximum(rho_rsT_ref[...] + rho_cs_ref[...], 0.0001)
    p_i = _pressure_from_rho(rho_i)
    p_j = _pressure_from_rho(rho_j)
    npi_term = -(p_i / (rho_i * rho_i))
    npj_term = -(p_j / (rho_j * rho_j))
    pref = (npi_term + npj_term) * cgrad
    ax = pref * dx
    ay = pref * dy

    visc_i = cr * ((2.0 * _NU) / rho_j)
    dvx = velT_ref[0:1, :] - vel_ref[:, 0:1]
    dvy = velT_ref[1:2, :] - vel_ref[:, 1:2]
    fx = jnp.where(mask, ax + dvx * visc_i, 0.0)
    fy = jnp.where(mask, ay + dvy * visc_i, 0.0)
    fsum = jnp.concatenate(
        [jnp.sum(fx, axis=1, keepdims=True), jnp.sum(fy, axis=1, keepdims=True)],
        axis=1)

    @pl.when((r == 0) & (c == 0))
    def _():
        fc_ref[...] = jnp.zeros_like(fc_ref)

    @pl.when(c == 0)
    def _():
        fr_ref[...] = fsum

    @pl.when((c > 0) & _active(r, c))
    def _():
        fr_ref[...] = fr_ref[...] + fsum
        visc_j = cr * ((2.0 * _NU) / rho_i)
        gx = jnp.where(mask, ax + dvx * visc_j, 0.0)
        gy = jnp.where(mask, ay + dvy * visc_j, 0.0)
        bj = jax.lax.rem(r + c, _NB)
        sl = pl.ds(bj * _BR, _BR)
        fc_ref[0:1, sl] = fc_ref[0:1, sl] - jnp.sum(gx, axis=0, keepdims=True)
        fc_ref[1:2, sl] = fc_ref[1:2, sl] - jnp.sum(gy, axis=0, keepdims=True)


def _finalize_body(pos_ref, vel_ref, fr_ref, fcT_ref, dt_ref,
                   pos_out_ref, vel_out_ref):
    f = fr_ref[...] + fcT_ref[...]
    dt_v = dt_ref[0, 0]
    new_vx = vel_ref[:, 0:1] + dt_v * f[:, 0:1]
    new_vy = vel_ref[:, 1:2] + dt_v * (f[:, 1:2] + _GRAV_Y)
    new_vel = jnp.concatenate([new_vx, new_vy], axis=1)
    vel_out_ref[...] = new_vel
    pos_out_ref[...] = pos_ref[...] + dt_v * new_vel


def _bj_map(r, c):
    return jnp.where(c == 0, r, jax.lax.rem(r + c, _NB))


@jax.jit
def kernel(pos, vel, dt):
    n = pos.shape[0]
    pos = pos.astype(jnp.float32)
    vel = vel.astype(jnp.float32)
    pos_t = pos.T
    vel_t = vel.T
    dt_arr = jnp.asarray(dt, jnp.float32).reshape(1, 1)

    rho_r, rho_c = pl.pallas_call(
        _density_body,
        grid=(_NB, _NCO),
        in_specs=[
            pl.BlockSpec((_BR, _DIM), lambda r, c: (r, 0)),
            pl.BlockSpec((_DIM, _BR), lambda r, c: (0, _bj_map(r, c))),
        ],
        out_specs=[
            pl.BlockSpec((_BR, 1), lambda r, c: (r, 0)),
            pl.BlockSpec((1, n), lambda r, c: (0, 0)),
        ],
        out_shape=[
            jax.ShapeDtypeStruct((n, 1), jnp.float32),
            jax.ShapeDtypeStruct((1, n), jnp.float32),
        ],
    )(pos, pos_t)

    rho_rT = rho_r.reshape(1, n)
    rho_cT = rho_c.reshape(n, 1)

    f_r, f_c = pl.pallas_call(
        _force_body,
        grid=(_NB, _NCO),
        in_specs=[
            pl.BlockSpec((_BR, _DIM), lambda r, c: (r, 0)),
            pl.BlockSpec((_BR, _DIM), lambda r, c: (r, 0)),
            pl.BlockSpec((_DIM, _BR), lambda r, c: (0, _bj_map(r, c))),
            pl.BlockSpec((_DIM, _BR), lambda r, c: (0, _bj_map(r, c))),
            pl.BlockSpec((_BR, 1), lambda r, c: (r, 0)),
            pl.BlockSpec((_BR, 1), lambda r, c: (r, 0)),
            pl.BlockSpec((1, _BR), lambda r, c: (0, _bj_map(r, c))),
            pl.BlockSpec((1, _BR), lambda r, c: (0, _bj_map(r, c))),
        ],
        out_specs=[
            pl.BlockSpec((_BR, _DIM), lambda r, c: (r, 0)),
            pl.BlockSpec((_DIM, n), lambda r, c: (0, 0)),
        ],
        out_shape=[
            jax.ShapeDtypeStruct((n, _DIM), jnp.float32),
            jax.ShapeDtypeStruct((_DIM, n), jnp.float32),
        ],
    )(pos, vel, pos_t, vel_t, rho_r, rho_cT, rho_rT, rho_c)

    f_cT = f_c.T

    new_pos, new_vel = pl.pallas_call(
        _finalize_body,
        grid=(_NB,),
        in_specs=[
            pl.BlockSpec((_BR, _DIM), lambda r: (r, 0)),
            pl.BlockSpec((_BR, _DIM), lambda r: (r, 0)),
            pl.BlockSpec((_BR, _DIM), lambda r: (r, 0)),
            pl.BlockSpec((_BR, _DIM), lambda r: (r, 0)),
            pl.BlockSpec((1, 1), lambda r: (0, 0)),
        ],
        out_specs=[
            pl.BlockSpec((_BR, _DIM), lambda r: (r, 0)),
            pl.BlockSpec((_BR, _DIM), lambda r: (r, 0)),
        ],
        out_shape=[
            jax.ShapeDtypeStruct((n, _DIM), jnp.float32),
            jax.ShapeDtypeStruct((n, _DIM), jnp.float32),
        ],
    )(pos, vel, f_r, f_cT, dt_arr)

    return (new_pos, new_vel)


# speedup vs baseline: 1.4222x; 1.4222x over previous
import jax
import jax.numpy as jnp
from jax.experimental import pallas as pl

_H = 0.3
_DIM = 2
_RHO0 = 1000.0
_C0 = 10.0
_NU = 0.0001
_GAMMA = 7.0
_B = _RHO0 * _C0 ** 2 / _GAMMA
_PI = 3.14159265
_SIGMA = 10.0 / (7.0 * _PI)
_ALPHA = _SIGMA / _H ** _DIM
_INV_H = 1.0 / _H
_GCOEF = _ALPHA / _H ** 2
_CG_A = 2.25 * _GCOEF * _INV_H
_CG_B = -3.0 * _GCOEF
_GRAV_Y = -9.81

_BR = 1024
_NB = 4
_NCO = _NB // 2 + 1


def _pressure_from_rho(rho):
    x = rho * (1.0 / _RHO0)
    x2 = x * x
    x3 = x2 * x
    return _B * (x3 * x3 * x - 1.0)


def _pair_geometry(pos_ref, posT_ref):
    x_i = pos_ref[:, 0:1]
    y_i = pos_ref[:, 1:2]
    x_j = posT_ref[0:1, :]
    y_j = posT_ref[1:2, :]
    dx = x_j - x_i
    dy = y_j - y_i
    d2 = dx * dx + dy * dy
    dist = jnp.sqrt(jnp.maximum(d2, 1e-24))
    return dx, dy, d2, dist


def _active(r, c):
    return jnp.logical_not((c == _NCO - 1) & (r >= _NB // 2))


def _density_body(pos_ref, posT_ref, rho_r_ref, rho_c_ref):
    r = pl.program_id(0)
    c = pl.program_id(1)
    _, _, _, dist = _pair_geometry(pos_ref, posT_ref)
    q = jnp.minimum(dist * _INV_H, 2.0)
    q2 = q * q
    w_in = _ALPHA + q2 * ((0.75 * _ALPHA) * q - (1.5 * _ALPHA))
    s = 2.0 - q
    w_out = (0.25 * _ALPHA) * (s * s) * s
    w = jnp.where(q < 1.0, w_in, w_out)
    row_part = jnp.sum(w, axis=1, keepdims=True)

    @pl.when((r == 0) & (c == 0))
    def _():
        rho_c_ref[...] = jnp.zeros_like(rho_c_ref)

    @pl.when(c == 0)
    def _():
        rho_r_ref[...] = row_part

    @pl.when((c > 0) & _active(r, c))
    def _():
        rho_r_ref[...] = rho_r_ref[...] + row_part
        bj = jax.lax.rem(r + c, _NB)
        col_part = jnp.sum(w, axis=0, keepdims=True)
        sl = pl.ds(bj * _BR, _BR)
        rho_c_ref[0:1, sl] = rho_c_ref[0:1, sl] + col_part


def _force_body(pos_ref, vel_ref, posT_ref, velT_ref,
                rho_rs_ref, rho_csT_ref, rho_rsT_ref, rho_cs_ref,
                fr_ref, fc_ref):
    r = pl.program_id(0)
    c = pl.program_id(1)
    dx, dy, d2, dist = _pair_geometry(pos_ref, posT_ref)
    mask = (dist < _H) & (dist > 1e-10)
    cgrad = _CG_A * dist + _CG_B
    cr = cgrad * jnp.minimum(d2 * 1e10, 1.0)

    rho_i = jnp.maximum(rho_rs_ref[...] + rho_csT_ref[...], 0.0001)
    rho_j = jnp.maximum(rho_rsT_ref[...] + rho_cs_ref[...], 0.0001)
    p_i = _pressure_from_rho(rho_i)
    p_j = _pressure_from_rho(rho_j)
    npi_term = -(p_i / (rho_i * rho_i))
    npj_term = -(p_j / (rho_j * rho_j))
    pref = (npi_term + npj_term) * cgrad
    ax = pref * dx
    ay = pref * dy

    visc_i = cr * ((2.0 * _NU) / rho_j)
    dvx = velT_ref[0:1, :] - vel_ref[:, 0:1]
    dvy = velT_ref[1:2, :] - vel_ref[:, 1:2]
    fx = jnp.where(mask, ax + dvx * visc_i, 0.0)
    fy = jnp.where(mask, ay + dvy * visc_i, 0.0)
    fsum = jnp.concatenate(
        [jnp.sum(fx, axis=1, keepdims=True), jnp.sum(fy, axis=1, keepdims=True)],
        axis=1)

    @pl.when((r == 0) & (c == 0))
    def _():
        fc_ref[...] = jnp.zeros_like(fc_ref)

    @pl.when(c == 0)
    def _():
        fr_ref[...] = fsum

    @pl.when((c > 0) & _active(r, c))
    def _():
        fr_ref[...] = fr_ref[...] + fsum
        visc_j = cr * ((2.0 * _NU) / rho_i)
        gx = jnp.where(mask, ax + dvx * visc_j, 0.0)
        gy = jnp.where(mask, ay + dvy * visc_j, 0.0)
        bj = jax.lax.rem(r + c, _NB)
        sl = pl.ds(bj * _BR, _BR)
        fc_ref[0:1, sl] = fc_ref[0:1, sl] - jnp.sum(gx, axis=0, keepdims=True)
        fc_ref[1:2, sl] = fc_ref[1:2, sl] - jnp.sum(gy, axis=0, keepdims=True)


def _finalize_body(pos_ref, vel_ref, fr_ref, fcT_ref, dt_ref,
                   pos_out_ref, vel_out_ref):
    f = fr_ref[...] + fcT_ref[...]
    dt_v = dt_ref[0, 0]
    new_vx = vel_ref[:, 0:1] + dt_v * f[:, 0:1]
    new_vy = vel_ref[:, 1:2] + dt_v * (f[:, 1:2] + _GRAV_Y)
    new_vel = jnp.concatenate([new_vx, new_vy], axis=1)
    vel_out_ref[...] = new_vel
    pos_out_ref[...] = pos_ref[...] + dt_v * new_vel


def _bj_map(r, c):
    return jnp.where(c == 0, r, jax.lax.rem(r + c, _NB))


@jax.jit
def kernel(pos, vel, dt):
    n = pos.shape[0]
    pos = pos.astype(jnp.float32)
    vel = vel.astype(jnp.float32)
    pos_t = pos.T
    vel_t = vel.T
    dt_arr = jnp.asarray(dt, jnp.float32).reshape(1, 1)

    rho_r, rho_c = pl.pallas_call(
        _density_body,
        grid=(_NB, _NCO),
        in_specs=[
            pl.BlockSpec((_BR, _DIM), lambda r, c: (r, 0)),
            pl.BlockSpec((_DIM, _BR), lambda r, c: (0, _bj_map(r, c))),
        ],
        out_specs=[
            pl.BlockSpec((_BR, 1), lambda r, c: (r, 0)),
            pl.BlockSpec((1, n), lambda r, c: (0, 0)),
        ],
        out_shape=[
            jax.ShapeDtypeStruct((n, 1), jnp.float32),
            jax.ShapeDtypeStruct((1, n), jnp.float32),
        ],
    )(pos, pos_t)

    rho_rT = rho_r.reshape(1, n)
    rho_cT = rho_c.reshape(n, 1)

    f_r, f_c = pl.pallas_call(
        _force_body,
        grid=(_NB, _NCO),
        in_specs=[
            pl.BlockSpec((_BR, _DIM), lambda r, c: (r, 0)),
            pl.BlockSpec((_BR, _DIM), lambda r, c: (r, 0)),
            pl.BlockSpec((_DIM, _BR), lambda r, c: (0, _bj_map(r, c))),
            pl.BlockSpec((_DIM, _BR), lambda r, c: (0, _bj_map(r, c))),
            pl.BlockSpec((_BR, 1), lambda r, c: (r, 0)),
            pl.BlockSpec((_BR, 1), lambda r, c: (r, 0)),
            pl.BlockSpec((1, _BR), lambda r, c: (0, _bj_map(r, c))),
            pl.BlockSpec((1, _BR), lambda r, c: (0, _bj_map(r, c))),
        ],
        out_specs=[
            pl.BlockSpec((_BR, _DIM), lambda r, c: (r, 0)),
            pl.BlockSpec((_DIM, n), lambda r, c: (0, 0)),
        ],
        out_shape=[
            jax.ShapeDtypeStruct((n, _DIM), jnp.float32),
            jax.ShapeDtypeStruct((_DIM, n), jnp.float32),
        ],
    )(pos, vel, pos_t, vel_t, rho_r, rho_cT, rho_rT, rho_c)

    f_cT = f_c.T

    new_pos, new_vel = pl.pallas_call(
        _finalize_body,
        grid=(_NB,),
        in_specs=[
            pl.BlockSpec((_BR, _DIM), lambda r: (r, 0)),
            pl.BlockSpec((_BR, _DIM), lambda r: (r, 0)),
            pl.BlockSpec((_BR, _DIM), lambda r: (r, 0)),
            pl.BlockSpec((_BR, _DIM), lambda r: (r, 0)),
            pl.BlockSpec((1, 1), lambda r: (0, 0)),
        ],
        out_specs=[
            pl.BlockSpec((_BR, _DIM), lambda r: (r, 0)),
            pl.BlockSpec((_BR, _DIM), lambda r: (r, 0)),
        ],
        out_shape=[
            jax.ShapeDtypeStruct((n, _DIM), jnp.float32),
            jax.ShapeDtypeStruct((n, _DIM), jnp.float32),
        ],
    )(pos, vel, f_r, f_cT, dt_arr)

    return (new_pos, new_vel)


# speedup vs baseline: 1.5697x; 1.1037x over previous
import jax
import jax.numpy as jnp
from jax.experimental import pallas as pl

_H = 0.3
_DIM = 2
_RHO0 = 1000.0
_C0 = 10.0
_NU = 0.0001
_GAMMA = 7.0
_B = _RHO0 * _C0 ** 2 / _GAMMA
_PI = 3.14159265
_SIGMA = 10.0 / (7.0 * _PI)
_ALPHA = _SIGMA / _H ** _DIM
_INV_H = 1.0 / _H
_GCOEF = _ALPHA / _H ** 2
_CG_A = 2.25 * _GCOEF * _INV_H
_CG_B = -3.0 * _GCOEF
_GRAV_Y = -9.81

_BR = 512
_NB = 8
_NCO = _NB // 2 + 1


def _pressure_from_rho(rho):
    x = rho * (1.0 / _RHO0)
    x2 = x * x
    x3 = x2 * x
    return _B * (x3 * x3 * x - 1.0)


def _pair_geometry(pos_ref, posT_ref):
    x_i = pos_ref[:, 0:1]
    y_i = pos_ref[:, 1:2]
    x_j = posT_ref[0:1, :]
    y_j = posT_ref[1:2, :]
    dx = x_j - x_i
    dy = y_j - y_i
    d2 = dx * dx + dy * dy
    dist = jnp.sqrt(d2)
    return dx, dy, d2, dist


def _active(r, c):
    return jnp.logical_not((c == _NCO - 1) & (r >= _NB // 2))


def _density_body(pos_ref, posT_ref, rho_r_ref, rho_c_ref):
    r = pl.program_id(0)
    c = pl.program_id(1)
    _, _, _, dist = _pair_geometry(pos_ref, posT_ref)
    q = jnp.minimum(dist * _INV_H, 2.0)
    q2 = q * q
    w_in = _ALPHA + q2 * ((0.75 * _ALPHA) * q - (1.5 * _ALPHA))
    s = 2.0 - q
    w_out = (0.25 * _ALPHA) * (s * s) * s
    w = jnp.where(q < 1.0, w_in, w_out)
    row_part = jnp.sum(w, axis=1, keepdims=True)

    @pl.when((r == 0) & (c == 0))
    def _():
        rho_c_ref[...] = jnp.zeros_like(rho_c_ref)

    @pl.when(c == 0)
    def _():
        rho_r_ref[...] = row_part

    @pl.when((c > 0) & _active(r, c))
    def _():
        rho_r_ref[...] = rho_r_ref[...] + row_part
        bj = jax.lax.rem(r + c, _NB)
        col_part = jnp.sum(w, axis=0, keepdims=True)
        sl = pl.ds(bj * _BR, _BR)
        rho_c_ref[0:1, sl] = rho_c_ref[0:1, sl] + col_part


def _force_body(pos_ref, vel_ref, posT_ref, velT_ref,
                rho_rs_ref, rho_csT_ref, rho_rsT_ref, rho_cs_ref,
                fr_ref, fc_ref):
    r = pl.program_id(0)
    c = pl.program_id(1)
    dx, dy, d2, dist = _pair_geometry(pos_ref, posT_ref)
    mask = (dist < _H) & (dist > 1e-10)
    cgrad = jnp.where(mask, _CG_A * dist + _CG_B, 0.0)
    cr = cgrad * jnp.minimum(d2 * 1e10, 1.0)

    rho_i = jnp.maximum(rho_rs_ref[...] + rho_csT_ref[...], 0.0001)
    rho_j = jnp.maximum(rho_rsT_ref[...] + rho_cs_ref[...], 0.0001)
    p_i = _pressure_from_rho(rho_i)
    p_j = _pressure_from_rho(rho_j)
    npi_term = -(p_i / (rho_i * rho_i))
    npj_term = -(p_j / (rho_j * rho_j))
    pref = (npi_term + npj_term) * cgrad
    ax = pref * dx
    ay = pref * dy

    visc_i = cr * ((2.0 * _NU) / rho_j)
    dvx = velT_ref[0:1, :] - vel_ref[:, 0:1]
    dvy = velT_ref[1:2, :] - vel_ref[:, 1:2]
    fx = ax + dvx * visc_i
    fy = ay + dvy * visc_i
    fsum = jnp.concatenate(
        [jnp.sum(fx, axis=1, keepdims=True), jnp.sum(fy, axis=1, keepdims=True)],
        axis=1)

    @pl.when((r == 0) & (c == 0))
    def _():
        fc_ref[...] = jnp.zeros_like(fc_ref)

    @pl.when(c == 0)
    def _():
        fr_ref[...] = fsum

    @pl.when((c > 0) & _active(r, c))
    def _():
        fr_ref[...] = fr_ref[...] + fsum
        visc_j = cr * ((2.0 * _NU) / rho_i)
        gx = ax + dvx * visc_j
        gy = ay + dvy * visc_j
        bj = jax.lax.rem(r + c, _NB)
        sl = pl.ds(bj * _BR, _BR)
        fc_ref[0:1, sl] = fc_ref[0:1, sl] - jnp.sum(gx, axis=0, keepdims=True)
        fc_ref[1:2, sl] = fc_ref[1:2, sl] - jnp.sum(gy, axis=0, keepdims=True)


def _finalize_body(pos_ref, vel_ref, fr_ref, fcT_ref, dt_ref,
                   pos_out_ref, vel_out_ref):
    f = fr_ref[...] + fcT_ref[...]
    dt_v = dt_ref[0, 0]
    new_vx = vel_ref[:, 0:1] + dt_v * f[:, 0:1]
    new_vy = vel_ref[:, 1:2] + dt_v * (f[:, 1:2] + _GRAV_Y)
    new_vel = jnp.concatenate([new_vx, new_vy], axis=1)
    vel_out_ref[...] = new_vel
    pos_out_ref[...] = pos_ref[...] + dt_v * new_vel


def _bj_map(r, c):
    return jnp.where(c == 0, r, jax.lax.rem(r + c, _NB))


@jax.jit
def kernel(pos, vel, dt):
    n = pos.shape[0]
    pos = pos.astype(jnp.float32)
    vel = vel.astype(jnp.float32)
    pos_t = pos.T
    vel_t = vel.T
    dt_arr = jnp.asarray(dt, jnp.float32).reshape(1, 1)

    rho_r, rho_c = pl.pallas_call(
        _density_body,
        grid=(_NB, _NCO),
        in_specs=[
            pl.BlockSpec((_BR, _DIM), lambda r, c: (r, 0)),
            pl.BlockSpec((_DIM, _BR), lambda r, c: (0, _bj_map(r, c))),
        ],
        out_specs=[
            pl.BlockSpec((_BR, 1), lambda r, c: (r, 0)),
            pl.BlockSpec((1, n), lambda r, c: (0, 0)),
        ],
        out_shape=[
            jax.ShapeDtypeStruct((n, 1), jnp.float32),
            jax.ShapeDtypeStruct((1, n), jnp.float32),
        ],
    )(pos, pos_t)

    rho_rT = rho_r.reshape(1, n)
    rho_cT = rho_c.reshape(n, 1)

    f_r, f_c = pl.pallas_call(
        _force_body,
        grid=(_NB, _NCO),
        in_specs=[
            pl.BlockSpec((_BR, _DIM), lambda r, c: (r, 0)),
            pl.BlockSpec((_BR, _DIM), lambda r, c: (r, 0)),
            pl.BlockSpec((_DIM, _BR), lambda r, c: (0, _bj_map(r, c))),
            pl.BlockSpec((_DIM, _BR), lambda r, c: (0, _bj_map(r, c))),
            pl.BlockSpec((_BR, 1), lambda r, c: (r, 0)),
            pl.BlockSpec((_BR, 1), lambda r, c: (r, 0)),
            pl.BlockSpec((1, _BR), lambda r, c: (0, _bj_map(r, c))),
            pl.BlockSpec((1, _BR), lambda r, c: (0, _bj_map(r, c))),
        ],
        out_specs=[
            pl.BlockSpec((_BR, _DIM), lambda r, c: (r, 0)),
            pl.BlockSpec((_DIM, n), lambda r, c: (0, 0)),
        ],
        out_shape=[
            jax.ShapeDtypeStruct((n, _DIM), jnp.float32),
            jax.ShapeDtypeStruct((_DIM, n), jnp.float32),
        ],
    )(pos, vel, pos_t, vel_t, rho_r, rho_cT, rho_rT, rho_c)

    f_cT = f_c.T

    new_pos, new_vel = pl.pallas_call(
        _finalize_body,
        grid=(_NB,),
        in_specs=[
            pl.BlockSpec((_BR, _DIM), lambda r: (r, 0)),
            pl.BlockSpec((_BR, _DIM), lambda r: (r, 0)),
            pl.BlockSpec((_BR, _DIM), lambda r: (r, 0)),
            pl.BlockSpec((_BR, _DIM), lambda r: (r, 0)),
            pl.BlockSpec((1, 1), lambda r: (0, 0)),
        ],
        out_specs=[
            pl.BlockSpec((_BR, _DIM), lambda r: (r, 0)),
            pl.BlockSpec((_BR, _DIM), lambda r: (r, 0)),
        ],
        out_shape=[
            jax.ShapeDtypeStruct((n, _DIM), jnp.float32),
            jax.ShapeDtypeStruct((n, _DIM), jnp.float32),
        ],
    )(pos, vel, f_r, f_cT, dt_arr)

    return (new_pos, new_vel)
